# Initial kernel scaffold; baseline (speedup 1.0000x reference)
import jax
import jax.numpy as jnp
from jax.experimental import pallas as pl


def kernel(node_features, edge_index, edge_features, head_ent_nodes, tail_ent_nodes, W_fc, b_fc, W_pred, b_pred):
    raise NotImplementedError("write your pallas kernel here")


# speedup vs baseline: 61.2790x; 61.2790x over previous
import functools

import jax
import jax.numpy as jnp
from jax import lax
from jax.experimental import pallas as pl
from jax.experimental.pallas import tpu as pltpu
from jax.experimental.pallas import tpu_sc as plsc

_NODE_DIM = 128
_CHUNK = 100
_CHUNKS_PER_W = 2
_N_WORKERS = 32


def _gather_rows_sc(table, idx2d):
    n_chunks = idx2d.shape[0]
    mesh = plsc.VectorSubcoreMesh(core_axis_name="c", subcore_axis_name="s")

    @functools.partial(
        pl.kernel,
        out_type=jax.ShapeDtypeStruct((n_chunks, _CHUNK, _NODE_DIM), jnp.float32),
        mesh=mesh,
        scratch_types=[
            pltpu.VMEM((_CHUNKS_PER_W, _CHUNK), jnp.int32),
            pltpu.VMEM((_CHUNKS_PER_W, _CHUNK, _NODE_DIM), jnp.float32),
            pltpu.SemaphoreType.DMA,
        ],
    )
    def gather_kernel(table_hbm, idx_hbm, out_hbm, idx_v, rows_v, sem):
        wid = lax.axis_index("s") * 2 + lax.axis_index("c")
        base = wid * _CHUNKS_PER_W
        pltpu.sync_copy(idx_hbm.at[pl.ds(base, _CHUNKS_PER_W)], idx_v)
        copies = [
            pltpu.async_copy(table_hbm.at[idx_v.at[j]], rows_v.at[j], sem)
            for j in range(_CHUNKS_PER_W)
        ]
        for cp in copies:
            cp.wait()
        pltpu.sync_copy(rows_v, out_hbm.at[pl.ds(base, _CHUNKS_PER_W)])

    return gather_kernel(table, idx2d)


def _predict_tc(rows, W_fc, b_fc2d, Wp_head, Wp_tail, b_pred2d):
    n_pairs = rows.shape[0] // 2

    def body(rows_ref, wfc_ref, bfc_ref, wph_ref, wpt_ref, bp_ref, out_ref):
        g = jnp.dot(rows_ref[...], wfc_ref[...],
                    preferred_element_type=jnp.float32)
        g = jnp.maximum(g + bfc_ref[...], 0.0)
        out_ref[...] = (
            jnp.dot(g[:n_pairs], wph_ref[...], preferred_element_type=jnp.float32)
            + jnp.dot(g[n_pairs:], wpt_ref[...], preferred_element_type=jnp.float32)
            + bp_ref[...]
        )

    return pl.pallas_call(
        body,
        out_shape=jax.ShapeDtypeStruct((n_pairs, b_pred2d.shape[1]), jnp.float32),
    )(rows, W_fc, b_fc2d, Wp_head, Wp_tail, b_pred2d)


def kernel(node_features, edge_index, edge_features, head_ent_nodes,
           tail_ent_nodes, W_fc, b_fc, W_pred, b_pred):
    del edge_index, edge_features
    B, P = head_ent_nodes.shape
    out_num = b_pred.shape[0]
    node_dim = W_fc.shape[1]

    idx2d = jnp.concatenate([head_ent_nodes, tail_ent_nodes], axis=0)

    rows = _gather_rows_sc(node_features, idx2d)
    rows = rows.reshape(2 * B * P, node_dim)

    out = _predict_tc(rows, W_fc, b_fc.reshape(1, node_dim),
                      W_pred[:node_dim], W_pred[node_dim:],
                      b_pred.reshape(1, out_num))
    return out.reshape(B, P, out_num)
